# baseline (speedup 1.0000x reference)
import jax
import jax.numpy as jnp
from jax import lax
from jax.experimental import pallas as pl
from jax.experimental.pallas import tpu as pltpu
from jax.experimental.pallas import tpu_sc as plsc

_B = 4096
_S, _L = 50, 128
_NC, _NS = 2, 16
_NW = _NC * _NS
_RPW = _B // _NW
_CH = 8
_NCHUNK = _RPW // _CH
_NBUF = 2


def _sc_roll_body(x_ref, o_ref, buf0, buf1, rs0, rs1, ws0, ws1):
    bufs, rsems, wsems = [buf0, buf1], [rs0, rs1], [ws0, ws1]
    wid = lax.axis_index("s") * _NC + lax.axis_index("c")
    base = wid * _RPW

    def issue_read(c):
        b = c % _NBUF
        s = base + c * _CH
        if c < _NCHUNK - 1:
            return [pltpu.async_copy(x_ref.at[pl.ds(s + 1, _CH)],
                                     bufs[b], rsems[b])]
        src2 = lax.rem(s + _CH, _B)
        return [
            pltpu.async_copy(x_ref.at[pl.ds(s + 1, _CH - 1)],
                             bufs[b].at[pl.ds(0, _CH - 1)], rsems[b]),
            pltpu.async_copy(x_ref.at[pl.ds(src2, 1)],
                             bufs[b].at[pl.ds(_CH - 1, 1)], rsems[b]),
        ]

    def issue_write(c):
        b = c % _NBUF
        s = base + c * _CH
        return [pltpu.async_copy(bufs[b], o_ref.at[pl.ds(s, _CH)], wsems[b])]

    reads, writes = {}, {}
    reads[0] = issue_read(0)
    for c in range(_NCHUNK):
        nxt = c + 1
        if nxt < _NCHUNK:
            if nxt >= _NBUF:
                for h in writes[nxt - _NBUF]:
                    h.wait()
            reads[nxt] = issue_read(nxt)
        for h in reads[c]:
            h.wait()
        writes[c] = issue_write(c)
    for c in range(_NCHUNK - _NBUF, _NCHUNK):
        for h in writes[c]:
            h.wait()


def kernel(x):
    return pl.kernel(
        _sc_roll_body,
        out_type=jax.ShapeDtypeStruct((_B, _S, _L), jnp.float32),
        mesh=plsc.VectorSubcoreMesh(core_axis_name="c", subcore_axis_name="s"),
        scratch_types=[pltpu.VMEM((_CH, _S, _L), jnp.float32)] * _NBUF
                      + [pltpu.SemaphoreType.DMA] * (2 * _NBUF),
        compiler_params=pltpu.CompilerParams(use_tc_tiling_on_sc=True),
    )(x)


# speedup vs baseline: 2.4984x; 2.4984x over previous
import jax
import jax.numpy as jnp
from jax import lax
from jax.experimental import pallas as pl
from jax.experimental.pallas import tpu as pltpu
from jax.experimental.pallas import tpu_sc as plsc

_B = 4096
_SL = 50
_L = 128
_NC, _NS = 2, 16
_NW = _NC * _NS
_IPW = _B // _NW
_HALO = 8
_J = 2
_NSTEP = _SL // _J
_NBUF = 3


def _sc_roll_body(x_ref, o_ref, *scratch):
    bufs = list(scratch[:_NBUF])
    rsems = list(scratch[_NBUF:2 * _NBUF])
    wsems = list(scratch[2 * _NBUF:])
    wid = lax.axis_index("s") * _NC + lax.axis_index("c")
    s0 = wid * _IPW
    h0 = lax.rem(s0 + _IPW, _B)

    def issue_read(g):
        b = g % _NBUF
        j = g * _J
        return [
            pltpu.async_copy(x_ref.at[pl.ds(j, _J), pl.ds(s0, _IPW)],
                             bufs[b].at[:, pl.ds(0, _IPW)], rsems[b]),
            pltpu.async_copy(x_ref.at[pl.ds(j, _J), pl.ds(h0, _HALO)],
                             bufs[b].at[:, pl.ds(_IPW, _HALO)], rsems[b]),
        ]

    def issue_write(g):
        b = g % _NBUF
        j = g * _J
        return [pltpu.async_copy(bufs[b].at[:, pl.ds(1, _IPW)],
                                 o_ref.at[pl.ds(j, _J), pl.ds(s0, _IPW)],
                                 wsems[b])]

    depth = _NBUF - 1
    reads, writes = {}, {}
    for g in range(min(depth, _NSTEP)):
        reads[g] = issue_read(g)
    for g in range(_NSTEP):
        nxt = g + depth
        if nxt < _NSTEP:
            if nxt >= _NBUF:
                for h in writes[nxt - _NBUF]:
                    h.wait()
            reads[nxt] = issue_read(nxt)
        for h in reads[g]:
            h.wait()
        writes[g] = issue_write(g)
    for g in range(max(_NSTEP - _NBUF, 0), _NSTEP):
        for h in writes[g]:
            h.wait()


def kernel(x):
    xt = jnp.transpose(x, (1, 0, 2))
    out_t = pl.kernel(
        _sc_roll_body,
        out_type=jax.ShapeDtypeStruct((_SL, _B, _L), jnp.float32),
        mesh=plsc.VectorSubcoreMesh(core_axis_name="c", subcore_axis_name="s"),
        scratch_types=[pltpu.VMEM((_J, _IPW + _HALO, _L), jnp.float32)] * _NBUF
                      + [pltpu.SemaphoreType.DMA] * (2 * _NBUF),
    )(xt)
    return jnp.transpose(out_t, (1, 0, 2))
